# baseline (speedup 1.0000x reference)
import functools

import jax
import jax.numpy as jnp
from jax import lax
from jax.experimental import pallas as pl
from jax.experimental.pallas import tpu as pltpu
from jax.experimental.pallas import tpu_sc as plsc

_B = 4
_C = 128
_NH = 40962
_NL = 10242
_NLP = 10752
_NW = 32
_PT = _NLP // _NW
_CH = 16
_EPS = 1e-5

_mesh = plsc.VectorSubcoreMesh(core_axis_name="c", subcore_axis_name="s")


def _wid():
    return lax.axis_index("s") * 2 + lax.axis_index("c")


@functools.partial(
    pl.kernel,
    mesh=_mesh,
    out_type=jax.ShapeDtypeStruct((_B * _NLP, _C), jnp.float32),
    scratch_types=[
        pltpu.VMEM((_CH * 7,), jnp.int32),
        pltpu.VMEM((_CH * 7, _C), jnp.float32),
        pltpu.VMEM((_CH, _C), jnp.float32),
        pltpu.SemaphoreType.DMA,
    ],
)
def _pool_gather(table_hbm, idx_hbm, out_hbm, idx_v, rows_v, acc_v, sem):
    w = _wid()
    nchunks = (_B * _PT) // _CH

    def chunk(ci, _):
        base = w * (_B * _PT) + ci * _CH
        pltpu.sync_copy(idx_hbm.at[pl.ds(base * 7, _CH * 7)], idx_v)
        pltpu.async_copy(table_hbm.at[idx_v], rows_v, sem).wait()

        def node(n, _):
            r0 = n * 7
            for cg in range(_C // 16):
                sl = pl.ds(cg * 16, 16)
                a = rows_v[r0, sl]
                for j in range(1, 7):
                    a = a + rows_v[r0 + j, sl]
                acc_v[n, sl] = a * (1.0 / 7.0)
            return 0

        lax.fori_loop(0, _CH, node, 0)
        pltpu.sync_copy(acc_v, out_hbm.at[pl.ds(base, _CH)])
        return 0

    lax.fori_loop(0, nchunks, chunk, 0)


@functools.partial(
    pl.kernel,
    mesh=_mesh,
    out_type=jax.ShapeDtypeStruct((_B * _NLP * 7, _C), jnp.float32),
    scratch_types=[
        pltpu.VMEM((_CH * 7,), jnp.int32),
        pltpu.VMEM((_CH * 7, _C), jnp.float32),
        pltpu.SemaphoreType.DMA,
    ],
)
def _ring_gather(table_hbm, idx_hbm, out_hbm, idx_v, rows_v, sem):
    w = _wid()
    nchunks = (_B * _PT) // _CH

    def chunk(ci, _):
        base = w * (_B * _PT) + ci * _CH
        pltpu.sync_copy(idx_hbm.at[pl.ds(base * 7, _CH * 7)], idx_v)
        pltpu.async_copy(table_hbm.at[idx_v], rows_v, sem).wait()
        pltpu.sync_copy(rows_v, out_hbm.at[pl.ds(base * 7, _CH * 7)])
        return 0

    lax.fori_loop(0, nchunks, chunk, 0)


_BLK = 512
_NBLK = (_B * _NLP) // _BLK


def _mm_stats_body(mat_ref, w_ref, b1_ref, out_ref, st_ref):
    j = pl.program_id(0)
    o = (
        jnp.dot(mat_ref[...], w_ref[...], preferred_element_type=jnp.float32)
        + b1_ref[...]
    )
    out_ref[...] = o
    row = j * _BLK + lax.broadcasted_iota(jnp.int32, (_BLK, 1), 0)
    node = row % _NLP
    om = jnp.where(node < _NL, o, 0.0)

    @pl.when(j == 0)
    def _init():
        st_ref[...] = jnp.zeros_like(st_ref)

    st_ref[0:1, :] += jnp.sum(om, axis=0, keepdims=True)
    st_ref[1:2, :] += jnp.sum(om * om, axis=0, keepdims=True)


def _fuse_body(o_ref, x1_ref, sc_ref, sh_ref, wa_ref, wb_ref, bc_ref, y_ref):
    z = o_ref[...] * sc_ref[...] + sh_ref[...]
    z = jnp.where(z >= 0.0, z, 0.2 * z)
    y_ref[...] = (
        jnp.dot(z, wa_ref[...], preferred_element_type=jnp.float32)
        + jnp.dot(x1_ref[...], wb_ref[...], preferred_element_type=jnp.float32)
        + bc_ref[...]
    )


def kernel(x, x1, neigh_orders, pool_neigh_orders, W1, b1, gamma, beta, Wc, bc):
    B, C, Nh = x.shape
    Nl = (Nh + 6) // 4

    xT = jnp.transpose(x, (0, 2, 1)).reshape(B * Nh, C)
    pad = (0, (_NLP - Nl) * 7)
    boff = (jnp.arange(B, dtype=jnp.int32) * jnp.int32(Nh))[:, None]
    pool_all = (
        jnp.pad(pool_neigh_orders[: Nl * 7], pad)[None, :] + boff
    ).reshape(-1)
    boff_l = (jnp.arange(B, dtype=jnp.int32) * jnp.int32(_NLP))[:, None]
    neigh_all = (
        jnp.pad(neigh_orders[: Nl * 7], pad)[None, :] + boff_l
    ).reshape(-1)

    xp = _pool_gather(xT, pool_all)
    matg = _ring_gather(xp, neigh_all)
    matg = matg.reshape(_B * _NLP, 7 * C)

    outT, stats = pl.pallas_call(
        _mm_stats_body,
        grid=(_NBLK,),
        in_specs=[
            pl.BlockSpec((_BLK, 7 * C), lambda j: (j, 0)),
            pl.BlockSpec((7 * C, C), lambda j: (0, 0)),
            pl.BlockSpec((1, C), lambda j: (0, 0)),
        ],
        out_specs=[
            pl.BlockSpec((_BLK, C), lambda j: (j, 0)),
            pl.BlockSpec((8, C), lambda j: (0, 0)),
        ],
        out_shape=[
            jax.ShapeDtypeStruct((_B * _NLP, C), jnp.float32),
            jax.ShapeDtypeStruct((8, C), jnp.float32),
        ],
    )(matg, W1.T, b1[None, :])

    cnt = jnp.float32(B * Nl)
    mean = stats[0] / cnt
    var = stats[1] / cnt - mean * mean
    scale = gamma * lax.rsqrt(var + _EPS)
    shift = beta - mean * scale

    x1T = jnp.pad(
        jnp.transpose(x1, (0, 2, 1)), ((0, 0), (0, _NLP - Nl), (0, 0))
    ).reshape(B * _NLP, C)
    yT = pl.pallas_call(
        _fuse_body,
        grid=(_NBLK,),
        in_specs=[
            pl.BlockSpec((_BLK, C), lambda j: (j, 0)),
            pl.BlockSpec((_BLK, C), lambda j: (j, 0)),
            pl.BlockSpec((1, C), lambda j: (0, 0)),
            pl.BlockSpec((1, C), lambda j: (0, 0)),
            pl.BlockSpec((C, C), lambda j: (0, 0)),
            pl.BlockSpec((C, C), lambda j: (0, 0)),
            pl.BlockSpec((1, C), lambda j: (0, 0)),
        ],
        out_specs=pl.BlockSpec((_BLK, C), lambda j: (j, 0)),
        out_shape=jax.ShapeDtypeStruct((_B * _NLP, C), jnp.float32),
    )(
        outT,
        x1T,
        scale[None, :],
        shift[None, :],
        Wc[:, :C].T,
        Wc[:, C:].T,
        bc[None, :],
    )

    y = yT.reshape(B, _NLP, C)[:, :Nl, :]
    return jnp.transpose(y, (0, 2, 1))


# speedup vs baseline: 1.2081x; 1.2081x over previous
import functools

import jax
import jax.numpy as jnp
from jax import lax
from jax.experimental import pallas as pl
from jax.experimental.pallas import tpu as pltpu
from jax.experimental.pallas import tpu_sc as plsc

_B = 4
_C = 128
_NH = 40962
_NL = 10242
_NLP = 10752
_NW = 32
_PT = _NLP // _NW
_CH = 16
_EPS = 1e-5

_mesh = plsc.VectorSubcoreMesh(core_axis_name="c", subcore_axis_name="s")

_NCH = (_B * _PT) // _CH
_ROWS = _CH * 7
_NBUF = 4


def _wid():
    return lax.axis_index("s") * 2 + lax.axis_index("c")


@functools.partial(
    pl.kernel,
    mesh=_mesh,
    out_type=jax.ShapeDtypeStruct((_B * _NLP, _C), jnp.float32),
    scratch_types=[
        pltpu.VMEM((_NCH * _ROWS,), jnp.int32),
    ]
    + [pltpu.VMEM((_ROWS, _C), jnp.float32)] * _NBUF
    + [pltpu.VMEM((_CH, _C), jnp.float32)] * _NBUF
    + [pltpu.SemaphoreType.DMA] * (2 * _NBUF),
)
def _pool_gather(table_hbm, idx_hbm, out_hbm, idx_v, r0, r1, r2, r3,
                 o0, o1, o2, o3, g0, g1, g2, g3, w0, w1, w2, w3):
    w = _wid()
    rows = (r0, r1, r2, r3)
    outs = (o0, o1, o2, o3)
    gsem = (g0, g1, g2, g3)
    wsem = (w0, w1, w2, w3)
    pltpu.sync_copy(idx_hbm.at[pl.ds(w * _NCH * _ROWS, _NCH * _ROWS)], idx_v)

    def g_start(c, j):
        pltpu.async_copy(table_hbm.at[idx_v.at[pl.ds(c * _ROWS, _ROWS)]], rows[j], gsem[j])

    def g_wait(c, j):
        pltpu.make_async_copy(table_hbm.at[idx_v.at[pl.ds(c * _ROWS, _ROWS)]], rows[j], gsem[j]).wait()

    def wb_dst(c):
        return out_hbm.at[pl.ds(w * (_B * _PT) + c * _CH, _CH)]

    def wb_start(c, j):
        pltpu.async_copy(outs[j], wb_dst(c), wsem[j])

    def wb_wait(c, j):
        pltpu.make_async_copy(outs[j], wb_dst(c), wsem[j]).wait()

    def compute(j):
        rv, ov = rows[j], outs[j]

        def node(n, _):
            base = n * 7
            for cg in range(_C // 16):
                sl = pl.ds(cg * 16, 16)
                a = rv[base, sl]
                for t in range(1, 7):
                    a = a + rv[base + t, sl]
                ov[n, sl] = a * (1.0 / 7.0)
            return 0

        lax.fori_loop(0, _CH, node, 0)

    g_start(0, 0)
    g_start(1, 1)
    for jj in range(_NBUF):
        g_start(jj + 2, (jj + 2) % _NBUF)
        g_wait(jj, jj)
        compute(jj)
        wb_start(jj, jj)

    def blk(b, _):
        for jj in range(_NBUF):
            c = b * _NBUF + jj
            g_start(c + 2, (jj + 2) % _NBUF)
            g_wait(c, jj)
            wb_wait(c - _NBUF, jj)
            compute(jj)
            wb_start(c, jj)
        return 0

    lax.fori_loop(1, _NCH // _NBUF - 1, blk, 0)
    for jj in range(_NBUF):
        c = _NCH - _NBUF + jj
        if jj < 2:
            g_start(c + 2, (jj + 2) % _NBUF)
        g_wait(c, jj)
        wb_wait(c - _NBUF, jj)
        compute(jj)
        wb_start(c, jj)
    for jj in range(_NBUF):
        wb_wait(_NCH - _NBUF + jj, jj)


@functools.partial(
    pl.kernel,
    mesh=_mesh,
    out_type=jax.ShapeDtypeStruct((_B * _NLP * 7, _C), jnp.float32),
    scratch_types=[
        pltpu.VMEM((_NCH * _ROWS,), jnp.int32),
    ]
    + [pltpu.VMEM((_ROWS, _C), jnp.float32)] * _NBUF
    + [pltpu.SemaphoreType.DMA] * (2 * _NBUF),
)
def _ring_gather(table_hbm, idx_hbm, out_hbm, idx_v, r0, r1, r2, r3,
                 g0, g1, g2, g3, w0, w1, w2, w3):
    w = _wid()
    rows = (r0, r1, r2, r3)
    gsem = (g0, g1, g2, g3)
    wsem = (w0, w1, w2, w3)
    pltpu.sync_copy(idx_hbm.at[pl.ds(w * _NCH * _ROWS, _NCH * _ROWS)], idx_v)

    def g_start(c, j):
        pltpu.async_copy(table_hbm.at[idx_v.at[pl.ds(c * _ROWS, _ROWS)]], rows[j], gsem[j])

    def g_wait(c, j):
        pltpu.make_async_copy(table_hbm.at[idx_v.at[pl.ds(c * _ROWS, _ROWS)]], rows[j], gsem[j]).wait()

    def wb_dst(c):
        return out_hbm.at[pl.ds((w * (_B * _PT) + c * _CH) * 7, _ROWS)]

    def wb_start(c, j):
        pltpu.async_copy(rows[j], wb_dst(c), wsem[j])

    def wb_wait(c, j):
        pltpu.make_async_copy(rows[j], wb_dst(c), wsem[j]).wait()

    g_start(0, 0)
    g_start(1, 1)
    for jj in range(_NBUF):
        if jj >= 2:
            wb_wait(jj - 2, (jj + 2) % _NBUF)
        g_start(jj + 2, (jj + 2) % _NBUF)
        g_wait(jj, jj)
        wb_start(jj, jj)

    def blk(b, _):
        for jj in range(_NBUF):
            c = b * _NBUF + jj
            wb_wait(c - 2, (jj + 2) % _NBUF)
            g_start(c + 2, (jj + 2) % _NBUF)
            g_wait(c, jj)
            wb_start(c, jj)
        return 0

    lax.fori_loop(1, _NCH // _NBUF - 1, blk, 0)
    for jj in range(_NBUF):
        c = _NCH - _NBUF + jj
        if jj < 2:
            wb_wait(c - 2, (jj + 2) % _NBUF)
            g_start(c + 2, (jj + 2) % _NBUF)
        g_wait(c, jj)
        wb_start(c, jj)
    for jj in range(_NBUF):
        wb_wait(_NCH - _NBUF + jj, jj)


_BLK = 512
_NBLK = (_B * _NLP) // _BLK


def _mm_stats_body(mat_ref, w_ref, b1_ref, out_ref, st_ref):
    j = pl.program_id(0)
    o = (
        jnp.dot(mat_ref[...], w_ref[...], preferred_element_type=jnp.float32)
        + b1_ref[...]
    )
    out_ref[...] = o
    row = j * _BLK + lax.broadcasted_iota(jnp.int32, (_BLK, 1), 0)
    node = row % _NLP
    om = jnp.where(node < _NL, o, 0.0)

    @pl.when(j == 0)
    def _init():
        st_ref[...] = jnp.zeros_like(st_ref)

    st_ref[0:1, :] += jnp.sum(om, axis=0, keepdims=True)
    st_ref[1:2, :] += jnp.sum(om * om, axis=0, keepdims=True)


def _fuse_body(o_ref, x1_ref, sc_ref, sh_ref, wa_ref, wb_ref, bc_ref, y_ref):
    z = o_ref[...] * sc_ref[...] + sh_ref[...]
    z = jnp.where(z >= 0.0, z, 0.2 * z)
    y_ref[...] = (
        jnp.dot(z, wa_ref[...], preferred_element_type=jnp.float32)
        + jnp.dot(x1_ref[...], wb_ref[...], preferred_element_type=jnp.float32)
        + bc_ref[...]
    )


def kernel(x, x1, neigh_orders, pool_neigh_orders, W1, b1, gamma, beta, Wc, bc):
    B, C, Nh = x.shape
    Nl = (Nh + 6) // 4

    xT = jnp.transpose(x, (0, 2, 1)).reshape(B * Nh, C)
    pad = (0, (_NLP - Nl) * 7)
    boff = (jnp.arange(B, dtype=jnp.int32) * jnp.int32(Nh))[:, None]
    pool_all = (
        jnp.pad(pool_neigh_orders[: Nl * 7], pad)[None, :] + boff
    ).reshape(-1)
    boff_l = (jnp.arange(B, dtype=jnp.int32) * jnp.int32(_NLP))[:, None]
    neigh_all = (
        jnp.pad(neigh_orders[: Nl * 7], pad)[None, :] + boff_l
    ).reshape(-1)

    xp = _pool_gather(xT, pool_all)
    matg = _ring_gather(xp, neigh_all)
    matg = matg.reshape(_B * _NLP, 7 * C)

    outT, stats = pl.pallas_call(
        _mm_stats_body,
        grid=(_NBLK,),
        in_specs=[
            pl.BlockSpec((_BLK, 7 * C), lambda j: (j, 0)),
            pl.BlockSpec((7 * C, C), lambda j: (0, 0)),
            pl.BlockSpec((1, C), lambda j: (0, 0)),
        ],
        out_specs=[
            pl.BlockSpec((_BLK, C), lambda j: (j, 0)),
            pl.BlockSpec((8, C), lambda j: (0, 0)),
        ],
        out_shape=[
            jax.ShapeDtypeStruct((_B * _NLP, C), jnp.float32),
            jax.ShapeDtypeStruct((8, C), jnp.float32),
        ],
    )(matg, W1.T, b1[None, :])

    cnt = jnp.float32(B * Nl)
    mean = stats[0] / cnt
    var = stats[1] / cnt - mean * mean
    scale = gamma * lax.rsqrt(var + _EPS)
    shift = beta - mean * scale

    x1T = jnp.pad(
        jnp.transpose(x1, (0, 2, 1)), ((0, 0), (0, _NLP - Nl), (0, 0))
    ).reshape(B * _NLP, C)
    yT = pl.pallas_call(
        _fuse_body,
        grid=(_NBLK,),
        in_specs=[
            pl.BlockSpec((_BLK, C), lambda j: (j, 0)),
            pl.BlockSpec((_BLK, C), lambda j: (j, 0)),
            pl.BlockSpec((1, C), lambda j: (0, 0)),
            pl.BlockSpec((1, C), lambda j: (0, 0)),
            pl.BlockSpec((C, C), lambda j: (0, 0)),
            pl.BlockSpec((C, C), lambda j: (0, 0)),
            pl.BlockSpec((1, C), lambda j: (0, 0)),
        ],
        out_specs=pl.BlockSpec((_BLK, C), lambda j: (j, 0)),
        out_shape=jax.ShapeDtypeStruct((_B * _NLP, C), jnp.float32),
    )(
        outT,
        x1T,
        scale[None, :],
        shift[None, :],
        Wc[:, :C].T,
        Wc[:, C:].T,
        bc[None, :],
    )

    y = yT.reshape(B, _NLP, C)[:, :Nl, :]
    return jnp.transpose(y, (0, 2, 1))
